# baseline (speedup 1.0000x reference)
import functools

import jax
import jax.numpy as jnp
from jax import lax
from jax.experimental import pallas as pl
from jax.experimental.pallas import tpu as pltpu
from jax.experimental.pallas import tpu_sc as plsc

DIM = 64
NUM_CORES = 2
NUM_SUBCORES = 16
NUM_WORKERS = NUM_CORES * NUM_SUBCORES
CHUNK = 128


def kernel(x, embeddings):
    num_indices = x.size
    rows_per_worker = num_indices // NUM_WORKERS
    num_chunks = rows_per_worker // CHUNK
    indices = x.reshape(num_indices)
    mesh = plsc.VectorSubcoreMesh(core_axis_name="c", subcore_axis_name="s")

    @functools.partial(
        pl.kernel,
        mesh=mesh,
        compiler_params=pltpu.CompilerParams(use_tc_tiling_on_sc=False),
        out_type=jax.ShapeDtypeStruct((num_indices, DIM), embeddings.dtype),
        scratch_types=[
            pltpu.VMEM((rows_per_worker,), jnp.int32),
            pltpu.VMEM((CHUNK, DIM), embeddings.dtype),
            pltpu.SemaphoreType.DMA,
        ],
    )
    def gather_kernel(table_hbm, idx_hbm, out_hbm, idx_v, rows_v, sem):
        wid = lax.axis_index("s") * NUM_CORES + lax.axis_index("c")
        base = wid * rows_per_worker
        pltpu.sync_copy(idx_hbm.at[pl.ds(base, rows_per_worker)], idx_v)

        @pl.loop(0, num_chunks)
        def _(c):
            off = c * CHUNK
            pltpu.async_copy(
                table_hbm.at[idx_v.at[pl.ds(off, CHUNK)]], rows_v, sem
            ).wait()
            pltpu.sync_copy(rows_v, out_hbm.at[pl.ds(base + off, CHUNK)])

    out = gather_kernel(embeddings, indices)
    return out.reshape(x.shape + (DIM,))


# speedup vs baseline: 1.0463x; 1.0463x over previous
import functools

import jax
import jax.numpy as jnp
from jax import lax
from jax.experimental import pallas as pl
from jax.experimental.pallas import tpu as pltpu
from jax.experimental.pallas import tpu_sc as plsc

DIM = 64
NUM_CORES = 2
NUM_SUBCORES = 16
NUM_WORKERS = NUM_CORES * NUM_SUBCORES
CHUNK = 128
NBUF = 5


def kernel(x, embeddings):
    num_indices = x.size
    rows_per_worker = num_indices // NUM_WORKERS
    num_chunks = rows_per_worker // CHUNK
    indices = x.reshape(num_indices)
    mesh = plsc.VectorSubcoreMesh(core_axis_name="c", subcore_axis_name="s")

    @functools.partial(
        pl.kernel,
        mesh=mesh,
        compiler_params=pltpu.CompilerParams(use_tc_tiling_on_sc=False),
        out_type=jax.ShapeDtypeStruct((num_indices, DIM), embeddings.dtype),
        scratch_types=[
            pltpu.VMEM((rows_per_worker,), jnp.int32),
            pltpu.VMEM((NBUF, CHUNK, DIM), embeddings.dtype),
            pltpu.SemaphoreType.DMA((NBUF,)),
            pltpu.SemaphoreType.DMA((NBUF,)),
        ],
    )
    def gather_kernel(table_hbm, idx_hbm, out_hbm, idx_v, rows_v, gsem, ssem):
        wid = lax.axis_index("s") * NUM_CORES + lax.axis_index("c")
        base = wid * rows_per_worker
        pltpu.sync_copy(idx_hbm.at[pl.ds(base, rows_per_worker)], idx_v)

        def start_gather(c, b):
            pltpu.async_copy(
                table_hbm.at[idx_v.at[pl.ds(c * CHUNK, CHUNK)]],
                rows_v.at[b],
                gsem.at[b],
            )

        for b in range(NBUF):
            start_gather(b, b)

        @pl.loop(0, num_chunks, step=NBUF)
        def _(c0):
            for b in range(NBUF):
                c = c0 + b
                pltpu.make_async_copy(
                    table_hbm.at[idx_v.at[pl.ds(0, CHUNK)]],
                    rows_v.at[b],
                    gsem.at[b],
                ).wait()
                store = pltpu.async_copy(
                    rows_v.at[b], out_hbm.at[pl.ds(base + c * CHUNK, CHUNK)], ssem.at[b]
                )
                store.wait()

                @pl.when(c + NBUF < num_chunks)
                def _():
                    start_gather(c + NBUF, b)

    out = gather_kernel(embeddings, indices)
    return out.reshape(x.shape + (DIM,))


# speedup vs baseline: 1.1595x; 1.1082x over previous
import functools

import jax
import jax.numpy as jnp
from jax import lax
from jax.experimental import pallas as pl
from jax.experimental.pallas import tpu as pltpu
from jax.experimental.pallas import tpu_sc as plsc

DIM = 64
NUM_ROWS = 1000000
NUM_CORES = 2
NUM_SUBCORES = 16
NUM_WORKERS = NUM_CORES * NUM_SUBCORES
CHUNK = 128
NBUF = 5
TR = 2048
NUM_IN_BLOCKS = -(-NUM_ROWS // TR)
NUM_OUT_BLOCKS = -(-NUM_IN_BLOCKS // 2)
PAIR_ROWS = NUM_OUT_BLOCKS * TR


def _pair_transpose(emb_t):
    def body(lo_ref, hi_ref, out_ref):
        out_ref[...] = jnp.concatenate([lo_ref[...].T, hi_ref[...].T], axis=1)

    return pl.pallas_call(
        body,
        grid=(NUM_OUT_BLOCKS,),
        in_specs=[
            pl.BlockSpec((DIM, TR), lambda i: (0, 2 * i)),
            pl.BlockSpec(
                (DIM, TR), lambda i: (0, jnp.minimum(2 * i + 1, NUM_IN_BLOCKS - 1))
            ),
        ],
        out_specs=pl.BlockSpec((TR, 2 * DIM), lambda i: (i, 0)),
        out_shape=jax.ShapeDtypeStruct((PAIR_ROWS, 2 * DIM), emb_t.dtype),
        compiler_params=pltpu.CompilerParams(
            dimension_semantics=("parallel",),
        ),
    )(emb_t, emb_t)


def _sc_gather(table2, idx_pair):
    num_indices = idx_pair.shape[0]
    rows_per_worker = num_indices // NUM_WORKERS
    num_chunks = rows_per_worker // CHUNK
    mesh = plsc.VectorSubcoreMesh(core_axis_name="c", subcore_axis_name="s")

    @functools.partial(
        pl.kernel,
        mesh=mesh,
        out_type=jax.ShapeDtypeStruct((num_indices, 2 * DIM), table2.dtype),
        scratch_types=[
            pltpu.VMEM((rows_per_worker,), jnp.int32),
            pltpu.VMEM((NBUF, CHUNK, 2 * DIM), table2.dtype),
            pltpu.SemaphoreType.DMA((NBUF,)),
            pltpu.SemaphoreType.DMA((NBUF,)),
        ],
    )
    def gather_kernel(table_hbm, idx_hbm, out_hbm, idx_v, rows_v, gsem, ssem):
        wid = lax.axis_index("s") * NUM_CORES + lax.axis_index("c")
        base = wid * rows_per_worker
        pltpu.sync_copy(idx_hbm.at[pl.ds(base, rows_per_worker)], idx_v)

        def start_gather(c, b):
            pltpu.async_copy(
                table_hbm.at[idx_v.at[pl.ds(c * CHUNK, CHUNK)]],
                rows_v.at[b],
                gsem.at[b],
            )

        for b in range(NBUF):
            start_gather(b, b)

        @pl.loop(0, num_chunks, step=NBUF)
        def _(c0):
            for b in range(NBUF):
                c = c0 + b
                pltpu.make_async_copy(
                    table_hbm.at[idx_v.at[pl.ds(0, CHUNK)]],
                    rows_v.at[b],
                    gsem.at[b],
                ).wait()
                store = pltpu.async_copy(
                    rows_v.at[b], out_hbm.at[pl.ds(base + c * CHUNK, CHUNK)], ssem.at[b]
                )
                store.wait()

                @pl.when(c + NBUF < num_chunks)
                def _():
                    start_gather(c + NBUF, b)

    return gather_kernel(table2, idx_pair)


def kernel(x, embeddings):
    table2 = _pair_transpose(embeddings.T)
    idx = x.reshape(-1)
    idx_pair = ((idx >> 12) << 11) | (idx & (TR - 1))
    in_hi = (idx >> 11) & 1
    g = _sc_gather(table2, idx_pair)
    out = jnp.where(in_hi[:, None] == 1, g[:, DIM:], g[:, :DIM])
    return out.reshape(x.shape + (DIM,))


# speedup vs baseline: 1.8337x; 1.5814x over previous
import functools

import jax
import jax.numpy as jnp
from jax import lax
from jax.experimental import pallas as pl
from jax.experimental.pallas import tpu as pltpu
from jax.experimental.pallas import tpu_sc as plsc

DIM = 64
NUM_ROWS = 1000000
BATCH = 4096
SEQ = 50
NUM_CORES = 2
NUM_SUBCORES = 16
NUM_WORKERS = NUM_CORES * NUM_SUBCORES
CHUNK = 128
NBUF = 5
TR = 4096
TR_SHIFT = 12
NUM_IN_BLOCKS = -(-NUM_ROWS // TR)
NUM_OUT_BLOCKS = -(-NUM_IN_BLOCKS // 2)
PAIR_ROWS = NUM_OUT_BLOCKS * TR
SELB = 2048


def _pair_transpose(emb_t):
    def body(lo_ref, hi_ref, out_ref):
        out_ref[:, :DIM] = lo_ref[...].T
        out_ref[:, DIM:] = hi_ref[...].T

    in_cap = NUM_IN_BLOCKS - 1

    return pl.pallas_call(
        body,
        grid=(NUM_OUT_BLOCKS,),
        in_specs=[
            pl.BlockSpec((DIM, TR), lambda i: (0, jnp.minimum(2 * i, in_cap))),
            pl.BlockSpec((DIM, TR), lambda i: (0, jnp.minimum(2 * i + 1, in_cap))),
        ],
        out_specs=pl.BlockSpec((TR, 2 * DIM), lambda i: (i, 0)),
        out_shape=jax.ShapeDtypeStruct((PAIR_ROWS, 2 * DIM), emb_t.dtype),
        compiler_params=pltpu.CompilerParams(
            dimension_semantics=("arbitrary",),
        ),
    )(emb_t, emb_t)


def _sc_gather(table2, idx_pair):
    num_indices = idx_pair.shape[0]
    rows_per_worker = num_indices // NUM_WORKERS
    num_chunks = rows_per_worker // CHUNK
    mesh = plsc.VectorSubcoreMesh(core_axis_name="c", subcore_axis_name="s")

    @functools.partial(
        pl.kernel,
        mesh=mesh,
        out_type=jax.ShapeDtypeStruct((num_indices, 2 * DIM), table2.dtype),
        scratch_types=[
            pltpu.VMEM((rows_per_worker,), jnp.int32),
            pltpu.VMEM((NBUF, CHUNK, 2 * DIM), table2.dtype),
            pltpu.SemaphoreType.DMA((NBUF,)),
            pltpu.SemaphoreType.DMA((NBUF,)),
        ],
    )
    def gather_kernel(table_hbm, idx_hbm, out_hbm, idx_v, rows_v, gsem, ssem):
        wid = lax.axis_index("s") * NUM_CORES + lax.axis_index("c")
        base = wid * rows_per_worker
        pltpu.sync_copy(idx_hbm.at[pl.ds(base, rows_per_worker)], idx_v)

        def start_gather(c, b):
            pltpu.async_copy(
                table_hbm.at[idx_v.at[pl.ds(c * CHUNK, CHUNK)]],
                rows_v.at[b],
                gsem.at[b],
            )

        for b in range(NBUF):
            start_gather(b, b)

        @pl.loop(0, num_chunks, step=NBUF)
        def _(c0):
            for b in range(NBUF):
                c = c0 + b
                pltpu.make_async_copy(
                    table_hbm.at[idx_v.at[pl.ds(0, CHUNK)]],
                    rows_v.at[b],
                    gsem.at[b],
                ).wait()
                store = pltpu.async_copy(
                    rows_v.at[b], out_hbm.at[pl.ds(base + c * CHUNK, CHUNK)], ssem.at[b]
                )
                store.wait()

                @pl.when(c + NBUF < num_chunks)
                def _():
                    start_gather(c + NBUF, b)

    return gather_kernel(table2, idx_pair)


def _select_transpose(g2, x_t):
    def body(g_ref, x_ref, out_ref):
        gt = g_ref[...].T
        par = (x_ref[0] >> TR_SHIFT) & 1
        out_ref[0] = jnp.where(par == 1, gt[DIM:, :], gt[:DIM, :])

    x_t3 = x_t.reshape(SEQ, 1, BATCH)
    return pl.pallas_call(
        body,
        grid=(SEQ, BATCH // SELB),
        in_specs=[
            pl.BlockSpec((SELB, 2 * DIM), lambda s, c: (s * (BATCH // SELB) + c, 0)),
            pl.BlockSpec((1, 1, SELB), lambda s, c: (s, 0, c)),
        ],
        out_specs=pl.BlockSpec((1, DIM, SELB), lambda s, c: (s, 0, c)),
        out_shape=jax.ShapeDtypeStruct((SEQ, DIM, BATCH), g2.dtype),
        compiler_params=pltpu.CompilerParams(
            dimension_semantics=("arbitrary", "arbitrary"),
        ),
    )(g2, x_t3)


def kernel(x, embeddings):
    table2 = _pair_transpose(embeddings.T)
    x_t = x.T
    idx = x_t.reshape(-1)
    idx_pair = ((idx >> (TR_SHIFT + 1)) << TR_SHIFT) | (idx & (TR - 1))
    g2 = _sc_gather(table2, idx_pair)
    out_t = _select_transpose(g2, x_t)
    return jnp.transpose(out_t, (2, 0, 1))


# speedup vs baseline: 2.1363x; 1.1651x over previous
import functools

import jax
import jax.numpy as jnp
from jax import lax
from jax.experimental import pallas as pl
from jax.experimental.pallas import tpu as pltpu
from jax.experimental.pallas import tpu_sc as plsc

DIM = 64
NUM_ROWS = 1000000
BATCH = 4096
SEQ = 50
NUM_CORES = 2
NUM_SUBCORES = 16
NUM_WORKERS = NUM_CORES * NUM_SUBCORES
CHUNK = 128
NBUF = 5
TR = 4096
TR_SHIFT = 12
NUM_IN_BLOCKS = -(-NUM_ROWS // TR)
NUM_OUT_BLOCKS = -(-NUM_IN_BLOCKS // 2)
PAIR_ROWS = NUM_OUT_BLOCKS * TR
SELB = 2048


def _pair_transpose(emb_t):
    def body(lo_ref, hi_ref, out_ref):
        out_ref[...] = jnp.concatenate([lo_ref[...], hi_ref[...]], axis=0).T

    in_cap = NUM_IN_BLOCKS - 1

    return pl.pallas_call(
        body,
        grid=(NUM_OUT_BLOCKS,),
        in_specs=[
            pl.BlockSpec((DIM, TR), lambda i: (0, jnp.minimum(2 * i, in_cap))),
            pl.BlockSpec((DIM, TR), lambda i: (0, jnp.minimum(2 * i + 1, in_cap))),
        ],
        out_specs=pl.BlockSpec((TR, 2 * DIM), lambda i: (i, 0)),
        out_shape=jax.ShapeDtypeStruct((PAIR_ROWS, 2 * DIM), emb_t.dtype),
        compiler_params=pltpu.CompilerParams(
            dimension_semantics=("arbitrary",),
        ),
    )(emb_t, emb_t)


def _sc_gather(table2, idx_pair):
    num_indices = idx_pair.shape[0]
    rows_per_worker = num_indices // NUM_WORKERS
    num_chunks = rows_per_worker // CHUNK
    mesh = plsc.VectorSubcoreMesh(core_axis_name="c", subcore_axis_name="s")

    @functools.partial(
        pl.kernel,
        mesh=mesh,
        out_type=jax.ShapeDtypeStruct((num_indices, 2 * DIM), table2.dtype),
        scratch_types=[
            pltpu.VMEM((rows_per_worker,), jnp.int32),
            pltpu.VMEM((NBUF, CHUNK, 2 * DIM), table2.dtype),
            pltpu.SemaphoreType.DMA((NBUF,)),
            pltpu.SemaphoreType.DMA((NBUF,)),
        ],
    )
    def gather_kernel(table_hbm, idx_hbm, out_hbm, idx_v, rows_v, gsem, ssem):
        wid = lax.axis_index("s") * NUM_CORES + lax.axis_index("c")
        base = wid * rows_per_worker
        pltpu.sync_copy(idx_hbm.at[pl.ds(base, rows_per_worker)], idx_v)

        def start_gather(c, b):
            pltpu.async_copy(
                table_hbm.at[idx_v.at[pl.ds(c * CHUNK, CHUNK)]],
                rows_v.at[b],
                gsem.at[b],
            )

        for b in range(NBUF):
            start_gather(b, b)

        @pl.loop(0, num_chunks, step=NBUF)
        def _(c0):
            for b in range(NBUF):
                c = c0 + b
                pltpu.make_async_copy(
                    table_hbm.at[idx_v.at[pl.ds(0, CHUNK)]],
                    rows_v.at[b],
                    gsem.at[b],
                ).wait()
                store = pltpu.async_copy(
                    rows_v.at[b], out_hbm.at[pl.ds(base + c * CHUNK, CHUNK)], ssem.at[b]
                )
                store.wait()

                @pl.when(c + NBUF < num_chunks)
                def _():
                    start_gather(c + NBUF, b)

    return gather_kernel(table2, idx_pair)


def _select_transpose(g2, x_t):
    def body(g_ref, x_ref, out_ref):
        gt = g_ref[...].T
        par = (x_ref[0] >> TR_SHIFT) & 1
        out_ref[0] = jnp.where(par == 1, gt[DIM:, :], gt[:DIM, :])

    x_t3 = x_t.reshape(SEQ, 1, BATCH)
    return pl.pallas_call(
        body,
        grid=(SEQ, BATCH // SELB),
        in_specs=[
            pl.BlockSpec((SELB, 2 * DIM), lambda s, c: (s * (BATCH // SELB) + c, 0)),
            pl.BlockSpec((1, 1, SELB), lambda s, c: (s, 0, c)),
        ],
        out_specs=pl.BlockSpec((1, DIM, SELB), lambda s, c: (s, 0, c)),
        out_shape=jax.ShapeDtypeStruct((SEQ, DIM, BATCH), g2.dtype),
        compiler_params=pltpu.CompilerParams(
            dimension_semantics=("arbitrary", "arbitrary"),
        ),
    )(g2, x_t3)


def kernel(x, embeddings):
    table2 = _pair_transpose(embeddings.T)
    x_t = x.T
    idx = x_t.reshape(-1)
    idx_pair = ((idx >> (TR_SHIFT + 1)) << TR_SHIFT) | (idx & (TR - 1))
    g2 = _sc_gather(table2, idx_pair)
    out_t = _select_transpose(g2, x_t)
    return jnp.transpose(out_t, (2, 0, 1))


# speedup vs baseline: 2.4773x; 1.1596x over previous
import functools

import jax
import jax.numpy as jnp
from jax import lax
from jax.experimental import pallas as pl
from jax.experimental.pallas import tpu as pltpu
from jax.experimental.pallas import tpu_sc as plsc

DIM = 64
NUM_ROWS = 1000000
BATCH = 4096
SEQ = 50
NUM_CORES = 2
NUM_SUBCORES = 16
NUM_WORKERS = NUM_CORES * NUM_SUBCORES
CHUNK = 128
NBUF = 5
TR = 8192
TR_SHIFT = 13
NUM_IN_BLOCKS = -(-NUM_ROWS // TR)
NUM_OUT_BLOCKS = -(-NUM_IN_BLOCKS // 2)
PAIR_ROWS = NUM_OUT_BLOCKS * TR
SELB = 4096


def _pair_transpose(emb_t):
    def body(lo_ref, hi_ref, out_ref):
        out_ref[...] = jnp.concatenate([lo_ref[...], hi_ref[...]], axis=0).T

    in_cap = NUM_IN_BLOCKS - 1

    return pl.pallas_call(
        body,
        grid=(NUM_OUT_BLOCKS,),
        in_specs=[
            pl.BlockSpec((DIM, TR), lambda i: (0, jnp.minimum(2 * i, in_cap))),
            pl.BlockSpec((DIM, TR), lambda i: (0, jnp.minimum(2 * i + 1, in_cap))),
        ],
        out_specs=pl.BlockSpec((TR, 2 * DIM), lambda i: (i, 0)),
        out_shape=jax.ShapeDtypeStruct((PAIR_ROWS, 2 * DIM), emb_t.dtype),
        compiler_params=pltpu.CompilerParams(
            dimension_semantics=("arbitrary",),
        ),
    )(emb_t, emb_t)


def _sc_gather(table2, idx_pair):
    num_indices = idx_pair.shape[0]
    rows_per_worker = num_indices // NUM_WORKERS
    num_chunks = rows_per_worker // CHUNK
    mesh = plsc.VectorSubcoreMesh(core_axis_name="c", subcore_axis_name="s")

    @functools.partial(
        pl.kernel,
        mesh=mesh,
        out_type=jax.ShapeDtypeStruct((num_indices, 2 * DIM), table2.dtype),
        scratch_types=[
            pltpu.VMEM((rows_per_worker,), jnp.int32),
            pltpu.VMEM((NBUF, CHUNK, 2 * DIM), table2.dtype),
            pltpu.SemaphoreType.DMA((NBUF,)),
            pltpu.SemaphoreType.DMA((NBUF,)),
        ],
    )
    def gather_kernel(table_hbm, idx_hbm, out_hbm, idx_v, rows_v, gsem, ssem):
        wid = lax.axis_index("s") * NUM_CORES + lax.axis_index("c")
        base = wid * rows_per_worker
        pltpu.sync_copy(idx_hbm.at[pl.ds(base, rows_per_worker)], idx_v)

        def start_gather(c, b):
            pltpu.async_copy(
                table_hbm.at[idx_v.at[pl.ds(c * CHUNK, CHUNK)]],
                rows_v.at[b],
                gsem.at[b],
            )

        for b in range(NBUF):
            start_gather(b, b)

        @pl.loop(0, num_chunks, step=NBUF)
        def _(c0):
            for b in range(NBUF):
                c = c0 + b
                pltpu.make_async_copy(
                    table_hbm.at[idx_v.at[pl.ds(0, CHUNK)]],
                    rows_v.at[b],
                    gsem.at[b],
                ).wait()
                store = pltpu.async_copy(
                    rows_v.at[b], out_hbm.at[pl.ds(base + c * CHUNK, CHUNK)], ssem.at[b]
                )
                store.wait()

                @pl.when(c + NBUF < num_chunks)
                def _():
                    start_gather(c + NBUF, b)

    return gather_kernel(table2, idx_pair)


def _select_transpose(g2, x_t):
    def body(g_ref, x_ref, out_ref):
        gt = g_ref[...].T
        par = (x_ref[0] >> TR_SHIFT) & 1
        out_ref[0] = jnp.where(par == 1, gt[DIM:, :], gt[:DIM, :])

    x_t3 = x_t.reshape(SEQ, 1, BATCH)
    return pl.pallas_call(
        body,
        grid=(SEQ, BATCH // SELB),
        in_specs=[
            pl.BlockSpec((SELB, 2 * DIM), lambda s, c: (s * (BATCH // SELB) + c, 0)),
            pl.BlockSpec((1, 1, SELB), lambda s, c: (s, 0, c)),
        ],
        out_specs=pl.BlockSpec((1, DIM, SELB), lambda s, c: (s, 0, c)),
        out_shape=jax.ShapeDtypeStruct((SEQ, DIM, BATCH), g2.dtype),
        compiler_params=pltpu.CompilerParams(
            dimension_semantics=("arbitrary", "arbitrary"),
        ),
    )(g2, x_t3)


def kernel(x, embeddings):
    table2 = _pair_transpose(embeddings.T)
    x_t = x.T
    idx = x_t.reshape(-1)
    idx_pair = ((idx >> (TR_SHIFT + 1)) << TR_SHIFT) | (idx & (TR - 1))
    g2 = _sc_gather(table2, idx_pair)
    out_t = _select_transpose(g2, x_t)
    return jnp.transpose(out_t, (2, 0, 1))


# speedup vs baseline: 2.5256x; 1.0195x over previous
import functools

import jax
import jax.numpy as jnp
from jax import lax
from jax.experimental import pallas as pl
from jax.experimental.pallas import tpu as pltpu
from jax.experimental.pallas import tpu_sc as plsc

DIM = 64
NUM_ROWS = 1000000
BATCH = 4096
SEQ = 50
NUM_CORES = 2
NUM_SUBCORES = 16
NUM_WORKERS = NUM_CORES * NUM_SUBCORES
CHUNK = 128
NBUF = 5
TR = 16384
TR_SHIFT = 14
NUM_IN_BLOCKS = -(-NUM_ROWS // TR)
NUM_OUT_BLOCKS = -(-NUM_IN_BLOCKS // 2)
PAIR_ROWS = NUM_OUT_BLOCKS * TR
SELB = 4096


def _pair_transpose(emb_t):
    def body(lo_ref, hi_ref, out_ref):
        out_ref[...] = jnp.concatenate([lo_ref[...], hi_ref[...]], axis=0).T

    in_cap = NUM_IN_BLOCKS - 1

    return pl.pallas_call(
        body,
        grid=(NUM_OUT_BLOCKS,),
        in_specs=[
            pl.BlockSpec((DIM, TR), lambda i: (0, jnp.minimum(2 * i, in_cap))),
            pl.BlockSpec((DIM, TR), lambda i: (0, jnp.minimum(2 * i + 1, in_cap))),
        ],
        out_specs=pl.BlockSpec((TR, 2 * DIM), lambda i: (i, 0)),
        out_shape=jax.ShapeDtypeStruct((PAIR_ROWS, 2 * DIM), emb_t.dtype),
        compiler_params=pltpu.CompilerParams(
            dimension_semantics=("arbitrary",),
        ),
    )(emb_t, emb_t)


def _sc_gather(table2, idx_pair):
    num_indices = idx_pair.shape[0]
    rows_per_worker = num_indices // NUM_WORKERS
    num_chunks = rows_per_worker // CHUNK
    mesh = plsc.VectorSubcoreMesh(core_axis_name="c", subcore_axis_name="s")

    @functools.partial(
        pl.kernel,
        mesh=mesh,
        out_type=jax.ShapeDtypeStruct((num_indices, 2 * DIM), table2.dtype),
        scratch_types=[
            pltpu.VMEM((rows_per_worker,), jnp.int32),
            pltpu.VMEM((NBUF, CHUNK, 2 * DIM), table2.dtype),
            pltpu.SemaphoreType.DMA((NBUF,)),
            pltpu.SemaphoreType.DMA((NBUF,)),
        ],
    )
    def gather_kernel(table_hbm, idx_hbm, out_hbm, idx_v, rows_v, gsem, ssem):
        wid = lax.axis_index("s") * NUM_CORES + lax.axis_index("c")
        base = wid * rows_per_worker
        pltpu.sync_copy(idx_hbm.at[pl.ds(base, rows_per_worker)], idx_v)

        def start_gather(c, b):
            pltpu.async_copy(
                table_hbm.at[idx_v.at[pl.ds(c * CHUNK, CHUNK)]],
                rows_v.at[b],
                gsem.at[b],
            )

        for b in range(NBUF):
            start_gather(b, b)

        @pl.loop(0, num_chunks, step=NBUF)
        def _(c0):
            for b in range(NBUF):
                c = c0 + b
                pltpu.make_async_copy(
                    table_hbm.at[idx_v.at[pl.ds(0, CHUNK)]],
                    rows_v.at[b],
                    gsem.at[b],
                ).wait()
                store = pltpu.async_copy(
                    rows_v.at[b], out_hbm.at[pl.ds(base + c * CHUNK, CHUNK)], ssem.at[b]
                )
                store.wait()

                @pl.when(c + NBUF < num_chunks)
                def _():
                    start_gather(c + NBUF, b)

    return gather_kernel(table2, idx_pair)


def _select_transpose(g2, x_t):
    def body(g_ref, x_ref, out_ref):
        gt = g_ref[...].T
        par = (x_ref[0] >> TR_SHIFT) & 1
        out_ref[0] = jnp.where(par == 1, gt[DIM:, :], gt[:DIM, :])

    x_t3 = x_t.reshape(SEQ, 1, BATCH)
    return pl.pallas_call(
        body,
        grid=(SEQ, BATCH // SELB),
        in_specs=[
            pl.BlockSpec((SELB, 2 * DIM), lambda s, c: (s * (BATCH // SELB) + c, 0)),
            pl.BlockSpec((1, 1, SELB), lambda s, c: (s, 0, c)),
        ],
        out_specs=pl.BlockSpec((1, DIM, SELB), lambda s, c: (s, 0, c)),
        out_shape=jax.ShapeDtypeStruct((SEQ, DIM, BATCH), g2.dtype),
        compiler_params=pltpu.CompilerParams(
            dimension_semantics=("arbitrary", "arbitrary"),
        ),
    )(g2, x_t3)


def kernel(x, embeddings):
    table2 = _pair_transpose(embeddings.T)
    x_t = x.T
    idx = x_t.reshape(-1)
    idx_pair = ((idx >> (TR_SHIFT + 1)) << TR_SHIFT) | (idx & (TR - 1))
    g2 = _sc_gather(table2, idx_pair)
    out_t = _select_transpose(g2, x_t)
    return jnp.transpose(out_t, (2, 0, 1))


# speedup vs baseline: 2.6422x; 1.0462x over previous
import functools

import jax
import jax.numpy as jnp
from jax import lax
from jax.experimental import pallas as pl
from jax.experimental.pallas import tpu as pltpu
from jax.experimental.pallas import tpu_sc as plsc

DIM = 64
NUM_ROWS = 1000000
BATCH = 4096
SEQ = 50
NUM_CORES = 2
NUM_SUBCORES = 16
NUM_WORKERS = NUM_CORES * NUM_SUBCORES
CHUNK = 128
NBUF = 5
TR = 16384
TR_SHIFT = 14
NUM_IN_BLOCKS = -(-NUM_ROWS // TR)
NUM_OUT_BLOCKS = -(-NUM_IN_BLOCKS // 2)
PAIR_ROWS = NUM_OUT_BLOCKS * TR
SELS = 2


def _pair_transpose(emb_t):
    def body(lo_ref, hi_ref, out_ref):
        out_ref[...] = jnp.concatenate([lo_ref[...], hi_ref[...]], axis=0).T

    in_cap = NUM_IN_BLOCKS - 1

    return pl.pallas_call(
        body,
        grid=(NUM_OUT_BLOCKS,),
        in_specs=[
            pl.BlockSpec((DIM, TR), lambda i: (0, jnp.minimum(2 * i, in_cap))),
            pl.BlockSpec((DIM, TR), lambda i: (0, jnp.minimum(2 * i + 1, in_cap))),
        ],
        out_specs=pl.BlockSpec((TR, 2 * DIM), lambda i: (i, 0)),
        out_shape=jax.ShapeDtypeStruct((PAIR_ROWS, 2 * DIM), emb_t.dtype),
        compiler_params=pltpu.CompilerParams(
            dimension_semantics=("arbitrary",),
        ),
    )(emb_t, emb_t)


def _sc_gather(table2, idx_pair):
    num_indices = idx_pair.shape[0]
    rows_per_worker = num_indices // NUM_WORKERS
    num_chunks = rows_per_worker // CHUNK
    mesh = plsc.VectorSubcoreMesh(core_axis_name="c", subcore_axis_name="s")

    @functools.partial(
        pl.kernel,
        mesh=mesh,
        out_type=jax.ShapeDtypeStruct((num_indices, 2 * DIM), table2.dtype),
        scratch_types=[
            pltpu.VMEM((rows_per_worker,), jnp.int32),
            pltpu.VMEM((NBUF, CHUNK, 2 * DIM), table2.dtype),
            pltpu.SemaphoreType.DMA((NBUF,)),
            pltpu.SemaphoreType.DMA((NBUF,)),
        ],
    )
    def gather_kernel(table_hbm, idx_hbm, out_hbm, idx_v, rows_v, gsem, ssem):
        wid = lax.axis_index("s") * NUM_CORES + lax.axis_index("c")
        base = wid * rows_per_worker
        pltpu.sync_copy(idx_hbm.at[pl.ds(base, rows_per_worker)], idx_v)

        def start_gather(c, b):
            pltpu.async_copy(
                table_hbm.at[idx_v.at[pl.ds(c * CHUNK, CHUNK)]],
                rows_v.at[b],
                gsem.at[b],
            )

        for b in range(NBUF):
            start_gather(b, b)

        @pl.loop(0, num_chunks, step=NBUF)
        def _(c0):
            for b in range(NBUF):
                c = c0 + b
                pltpu.make_async_copy(
                    table_hbm.at[idx_v.at[pl.ds(0, CHUNK)]],
                    rows_v.at[b],
                    gsem.at[b],
                ).wait()
                store = pltpu.async_copy(
                    rows_v.at[b], out_hbm.at[pl.ds(base + c * CHUNK, CHUNK)], ssem.at[b]
                )
                store.wait()

                @pl.when(c + NBUF < num_chunks)
                def _():
                    start_gather(c + NBUF, b)

    return gather_kernel(table2, idx_pair)


def _select_transpose(g2, x_t):
    def body(g_ref, x_ref, out_ref):
        for k in range(SELS):
            gt = g_ref[pl.ds(k * BATCH, BATCH), :].T
            par = (x_ref[k] >> TR_SHIFT) & 1
            out_ref[k] = jnp.where(par == 1, gt[DIM:, :], gt[:DIM, :])

    x_t3 = x_t.reshape(SEQ, 1, BATCH)
    return pl.pallas_call(
        body,
        grid=(SEQ // SELS,),
        in_specs=[
            pl.BlockSpec((SELS * BATCH, 2 * DIM), lambda s: (s, 0)),
            pl.BlockSpec((SELS, 1, BATCH), lambda s: (s, 0, 0)),
        ],
        out_specs=pl.BlockSpec((SELS, DIM, BATCH), lambda s: (s, 0, 0)),
        out_shape=jax.ShapeDtypeStruct((SEQ, DIM, BATCH), g2.dtype),
        compiler_params=pltpu.CompilerParams(
            dimension_semantics=("arbitrary",),
        ),
    )(g2, x_t3)


def kernel(x, embeddings):
    table2 = _pair_transpose(embeddings.T)
    x_t = x.T
    idx = x_t.reshape(-1)
    idx_pair = ((idx >> (TR_SHIFT + 1)) << TR_SHIFT) | (idx & (TR - 1))
    g2 = _sc_gather(table2, idx_pair)
    out_t = _select_transpose(g2, x_t)
    return jnp.transpose(out_t, (2, 0, 1))


# speedup vs baseline: 2.6784x; 1.0137x over previous
import functools

import jax
import jax.numpy as jnp
from jax import lax
from jax.experimental import pallas as pl
from jax.experimental.pallas import tpu as pltpu
from jax.experimental.pallas import tpu_sc as plsc

DIM = 64
NUM_ROWS = 1000000
BATCH = 4096
SEQ = 50
NUM_CORES = 2
NUM_SUBCORES = 16
NUM_WORKERS = NUM_CORES * NUM_SUBCORES
CHUNK = 64
NBUF = 10
TR = 16384
TR_SHIFT = 14
NUM_IN_BLOCKS = -(-NUM_ROWS // TR)
NUM_OUT_BLOCKS = -(-NUM_IN_BLOCKS // 2)
PAIR_ROWS = NUM_OUT_BLOCKS * TR
SELS = 5


def _pair_transpose(emb_t):
    def body(lo_ref, hi_ref, out_ref):
        out_ref[...] = jnp.concatenate([lo_ref[...], hi_ref[...]], axis=0).T

    in_cap = NUM_IN_BLOCKS - 1

    return pl.pallas_call(
        body,
        grid=(NUM_OUT_BLOCKS,),
        in_specs=[
            pl.BlockSpec((DIM, TR), lambda i: (0, jnp.minimum(2 * i, in_cap))),
            pl.BlockSpec((DIM, TR), lambda i: (0, jnp.minimum(2 * i + 1, in_cap))),
        ],
        out_specs=pl.BlockSpec((TR, 2 * DIM), lambda i: (i, 0)),
        out_shape=jax.ShapeDtypeStruct((PAIR_ROWS, 2 * DIM), emb_t.dtype),
        compiler_params=pltpu.CompilerParams(
            dimension_semantics=("arbitrary",),
        ),
    )(emb_t, emb_t)


def _sc_gather(table2, idx_pair):
    num_indices = idx_pair.shape[0]
    rows_per_worker = num_indices // NUM_WORKERS
    num_chunks = rows_per_worker // CHUNK
    mesh = plsc.VectorSubcoreMesh(core_axis_name="c", subcore_axis_name="s")

    @functools.partial(
        pl.kernel,
        mesh=mesh,
        out_type=jax.ShapeDtypeStruct((num_indices, 2 * DIM), table2.dtype),
        scratch_types=[
            pltpu.VMEM((rows_per_worker,), jnp.int32),
            pltpu.VMEM((NBUF, CHUNK, 2 * DIM), table2.dtype),
            pltpu.SemaphoreType.DMA((NBUF,)),
            pltpu.SemaphoreType.DMA((NBUF,)),
        ],
    )
    def gather_kernel(table_hbm, idx_hbm, out_hbm, idx_v, rows_v, gsem, ssem):
        wid = lax.axis_index("s") * NUM_CORES + lax.axis_index("c")
        base = wid * rows_per_worker
        pltpu.sync_copy(idx_hbm.at[pl.ds(base, rows_per_worker)], idx_v)

        def start_gather(c, b):
            pltpu.async_copy(
                table_hbm.at[idx_v.at[pl.ds(c * CHUNK, CHUNK)]],
                rows_v.at[b],
                gsem.at[b],
            )

        for b in range(NBUF):
            start_gather(b, b)

        @pl.loop(0, num_chunks, step=NBUF)
        def _(c0):
            for b in range(NBUF):
                c = c0 + b
                pltpu.make_async_copy(
                    table_hbm.at[idx_v.at[pl.ds(0, CHUNK)]],
                    rows_v.at[b],
                    gsem.at[b],
                ).wait()
                store = pltpu.async_copy(
                    rows_v.at[b], out_hbm.at[pl.ds(base + c * CHUNK, CHUNK)], ssem.at[b]
                )
                store.wait()

                @pl.when(c + NBUF < num_chunks)
                def _():
                    start_gather(c + NBUF, b)

    return gather_kernel(table2, idx_pair)


def _select_transpose(g2, x_t):
    def body(g_ref, x_ref, out_ref):
        for k in range(SELS):
            gt = g_ref[pl.ds(k * BATCH, BATCH), :].T
            par = (x_ref[k] >> TR_SHIFT) & 1
            out_ref[k] = jnp.where(par == 1, gt[DIM:, :], gt[:DIM, :])

    x_t3 = x_t.reshape(SEQ, 1, BATCH)
    return pl.pallas_call(
        body,
        grid=(SEQ // SELS,),
        in_specs=[
            pl.BlockSpec((SELS * BATCH, 2 * DIM), lambda s: (s, 0)),
            pl.BlockSpec((SELS, 1, BATCH), lambda s: (s, 0, 0)),
        ],
        out_specs=pl.BlockSpec((SELS, DIM, BATCH), lambda s: (s, 0, 0)),
        out_shape=jax.ShapeDtypeStruct((SEQ, DIM, BATCH), g2.dtype),
        compiler_params=pltpu.CompilerParams(
            dimension_semantics=("arbitrary",),
        ),
    )(g2, x_t3)


def kernel(x, embeddings):
    table2 = _pair_transpose(embeddings.T)
    x_t = x.T
    idx = x_t.reshape(-1)
    idx_pair = ((idx >> (TR_SHIFT + 1)) << TR_SHIFT) | (idx & (TR - 1))
    g2 = _sc_gather(table2, idx_pair)
    out_t = _select_transpose(g2, x_t)
    return jnp.transpose(out_t, (2, 0, 1))
